# baseline (speedup 1.0000x reference)
import functools

import jax
import jax.numpy as jnp
from jax import lax
from jax.experimental import pallas as pl
from jax.experimental.pallas import tpu as pltpu, tpu_sc as plsc


def _gather_kernel(B, O, D, n_workers, num_cores):
    rows_per_w = B // n_workers
    mesh = plsc.VectorSubcoreMesh(core_axis_name="c", subcore_axis_name="s")

    @functools.partial(
        pl.kernel,
        mesh=mesh,
        out_type=jax.ShapeDtypeStruct((B, O, D), jnp.float32),
        scratch_types=[
            pltpu.VMEM((O,), jnp.int32),
            pltpu.VMEM((O,), jnp.int32),
            pltpu.VMEM((O, D), jnp.float32),
            pltpu.VMEM((O, D), jnp.float32),
            pltpu.SemaphoreType.DMA,
            pltpu.SemaphoreType.DMA,
            pltpu.SemaphoreType.DMA,
            pltpu.SemaphoreType.DMA,
        ],
        compiler_params=pltpu.CompilerParams(use_tc_tiling_on_sc=False),
    )
    def k(table_hbm, idx_hbm, out_hbm,
          idx_v0, idx_v1, rows_v0, rows_v1,
          sem_g0, sem_g1, sem_w0, sem_w1):
        idx_v = (idx_v0, idx_v1)
        rows_v = (rows_v0, rows_v1)
        sem_g = (sem_g0, sem_g1)
        sem_w = (sem_w0, sem_w1)

        wid = lax.axis_index("s") * num_cores + lax.axis_index("c")
        base = wid * rows_per_w

        def gather_copy(b):
            return pltpu.make_async_copy(
                table_hbm.at[idx_v[b]], rows_v[b], sem_g[b])

        def write_copy(b, row):
            return pltpu.make_async_copy(
                rows_v[b], out_hbm.at[row], sem_w[b])

        for b in range(2):
            pltpu.sync_copy(idx_hbm.at[base + b], idx_v[b])
            gather_copy(b).start()

        def body(t, carry):
            for b in range(2):
                row = base + 2 * t + b
                gather_copy(b).wait()
                write_copy(b, row).start()
                pltpu.sync_copy(idx_hbm.at[row + 2], idx_v[b])
                write_copy(b, row).wait()
                gather_copy(b).start()
            return carry

        lax.fori_loop(0, rows_per_w // 2 - 1, body, 0)

        for b in range(2):
            row = base + rows_per_w - 2 + b
            gather_copy(b).wait()
            write_copy(b, row).start()
        for b in range(2):
            row = base + rows_per_w - 2 + b
            write_copy(b, row).wait()

    return k


def kernel(overlap, scene, embed_table):
    B, O = overlap.shape
    V, D = embed_table.shape
    idx = overlap.astype(jnp.int32)

    info = plsc.get_sparse_core_info()
    n_workers = info.num_cores * info.num_subcores

    k = _gather_kernel(B, O, D, n_workers, info.num_cores)
    return k(embed_table, idx)


# speedup vs baseline: 1.3849x; 1.3849x over previous
import functools

import jax
import jax.numpy as jnp
from jax import lax
from jax.experimental import pallas as pl
from jax.experimental.pallas import tpu as pltpu, tpu_sc as plsc

_LANE = 16


def _gather_kernel(B, O, D, V, n_workers, num_cores):
    DT = D // 8
    BT = B // 128
    bt_per_w = BT // n_workers
    mesh = plsc.VectorSubcoreMesh(core_axis_name="c", subcore_axis_name="s")

    @functools.partial(
        pl.kernel,
        mesh=mesh,
        out_type=jax.ShapeDtypeStruct((O, DT, BT, 8, 128), jnp.float32),
        scratch_types=[
            pltpu.VMEM((O, 128), jnp.int32),
            pltpu.VMEM((8, V), jnp.float32),
            pltpu.VMEM((8, 128), jnp.float32),
            pltpu.VMEM((8, 128), jnp.float32),
            pltpu.SemaphoreType.DMA,
            pltpu.SemaphoreType.DMA,
        ],
        compiler_params=pltpu.CompilerParams(
            use_tc_tiling_on_sc=False, needs_layout_passes=False),
    )
    def k(tableT_hbm, idxT_hbm, out_hbm,
          idx_v, t8_v, tile0, tile1, sem_w0, sem_w1):
        tiles = (tile0, tile1)
        sems = (sem_w0, sem_w1)

        wid = lax.axis_index("s") * num_cores + lax.axis_index("c")

        r_vecs = [jnp.full((_LANE,), r, jnp.int32) for r in range(8)]

        def fill_tile(b, d1):
            for lb in range(128 // _LANE):
                iv = idx_v[d1, pl.ds(lb * _LANE, _LANE)]
                for r in range(8):
                    val = plsc.load_gather(t8_v, [r_vecs[r], iv])
                    tiles[b][r, pl.ds(lb * _LANE, _LANE)] = val

        def write_copy(b, d1, d2t, bt):
            return pltpu.make_async_copy(
                tiles[b], out_hbm.at[d1, d2t, bt], sems[b])

        def body_bt(j, carry):
            bt = wid * bt_per_w + j
            pltpu.sync_copy(idxT_hbm.at[:, pl.ds(bt * 128, 128)], idx_v)

            def body_d2t(d2t, carry2):
                pltpu.sync_copy(tableT_hbm.at[pl.ds(d2t * 8, 8)], t8_v)

                for b in range(2):
                    fill_tile(b, b)
                    write_copy(b, b, d2t, bt).start()

                def body_d1(t, carry3):
                    for b in range(2):
                        d1 = 2 * t + b
                        write_copy(b, d1 - 2, d2t, bt).wait()
                        fill_tile(b, d1)
                        write_copy(b, d1, d2t, bt).start()
                    return carry3

                lax.fori_loop(1, O // 2, body_d1, 0)

                for b in range(2):
                    write_copy(b, O - 2 + b, d2t, bt).wait()
                return carry2

            lax.fori_loop(0, DT, body_d2t, 0)
            return carry

        lax.fori_loop(0, bt_per_w, body_bt, 0)

    return k


def kernel(overlap, scene, embed_table):
    B, O = overlap.shape
    V, D = embed_table.shape
    idx_T = overlap.T.astype(jnp.int32)
    table_T = embed_table.T

    info = plsc.get_sparse_core_info()
    n_workers = info.num_cores * info.num_subcores

    k = _gather_kernel(B, O, D, V, n_workers, info.num_cores)
    out5 = k(table_T, idx_T)
    return out5.transpose((2, 4, 0, 1, 3)).reshape(B, O, D)


# speedup vs baseline: 4.3987x; 3.1762x over previous
import functools

import jax
import jax.numpy as jnp
from jax import lax
from jax.experimental import pallas as pl
from jax.experimental.pallas import tpu as pltpu, tpu_sc as plsc

_LANE = 16


def _gather_kernel(B, O, D, V, n_workers, num_cores):
    DT = D // 8
    BT = B // 128
    bt_per_w = BT // n_workers
    mesh = plsc.VectorSubcoreMesh(core_axis_name="c", subcore_axis_name="s")

    @functools.partial(
        pl.kernel,
        mesh=mesh,
        out_type=jax.ShapeDtypeStruct((O, DT, BT, 8, 128), jnp.float32),
        scratch_types=[
            pltpu.VMEM((O, 128), jnp.int32),
            pltpu.VMEM((8, V), jnp.float32),
            pltpu.VMEM((8, 128), jnp.float32),
            pltpu.VMEM((8, 128), jnp.float32),
            pltpu.SemaphoreType.DMA,
            pltpu.SemaphoreType.DMA,
        ],
        compiler_params=pltpu.CompilerParams(
            use_tc_tiling_on_sc=False, needs_layout_passes=False),
    )
    def k(tableT_hbm, idxT_hbm, out_hbm,
          idx_v, t8_v, tile0, tile1, sem_w0, sem_w1):
        tiles = (tile0, tile1)
        sems = (sem_w0, sem_w1)

        wid = lax.axis_index("s") * num_cores + lax.axis_index("c")

        t8_rows = [t8_v.at[r] for r in range(8)]

        def fill_tile(b, d1):
            nlb = 128 // _LANE
            ivs = [idx_v[d1, pl.ds(lb * _LANE, _LANE)] for lb in range(nlb)]
            prev = [plsc.load_gather(t8_rows[r], [ivs[0]]) for r in range(8)]
            for lb in range(1, nlb):
                cur = []
                for r in range(8):
                    cur.append(plsc.load_gather(t8_rows[r], [ivs[lb]]))
                    tiles[b][r, pl.ds((lb - 1) * _LANE, _LANE)] = prev[r]
                prev = cur
            for r in range(8):
                tiles[b][r, pl.ds((nlb - 1) * _LANE, _LANE)] = prev[r]

        def write_copy(b, d1, d2t, bt):
            return pltpu.make_async_copy(
                tiles[b], out_hbm.at[d1, d2t, bt], sems[b])

        def body_bt(j, carry):
            bt = wid * bt_per_w + j
            pltpu.sync_copy(idxT_hbm.at[:, pl.ds(bt * 128, 128)], idx_v)

            def body_d2t(d2t, carry2):
                pltpu.sync_copy(tableT_hbm.at[pl.ds(d2t * 8, 8)], t8_v)

                for b in range(2):
                    fill_tile(b, b)
                    write_copy(b, b, d2t, bt).start()

                def body_d1(t, carry3):
                    for b in range(2):
                        d1 = 2 * t + b
                        write_copy(b, d1 - 2, d2t, bt).wait()
                        fill_tile(b, d1)
                        write_copy(b, d1, d2t, bt).start()
                    return carry3

                lax.fori_loop(1, O // 2, body_d1, 0)

                for b in range(2):
                    write_copy(b, O - 2 + b, d2t, bt).wait()
                return carry2

            lax.fori_loop(0, DT, body_d2t, 0)
            return carry

        lax.fori_loop(0, bt_per_w, body_bt, 0)

    return k


def kernel(overlap, scene, embed_table):
    B, O = overlap.shape
    V, D = embed_table.shape
    idx_T = overlap.T.astype(jnp.int32)
    table_T = embed_table.T

    info = plsc.get_sparse_core_info()
    n_workers = info.num_cores * info.num_subcores

    k = _gather_kernel(B, O, D, V, n_workers, info.num_cores)
    out5 = k(table_T, idx_T)
    return out5.transpose((2, 4, 0, 1, 3)).reshape(B, O, D)


# speedup vs baseline: 6.3180x; 1.4363x over previous
import functools

import jax
import jax.numpy as jnp
from jax import lax
from jax.experimental import pallas as pl
from jax.experimental.pallas import tpu as pltpu, tpu_sc as plsc

_LANE = 16


def _gather_kernel(B, O, D, V, n_workers, num_cores):
    DT = D // 8
    BT = B // 128
    HG = 2
    DTH = DT // HG
    bt_per_w = BT // n_workers
    nlb = 128 // _LANE
    mesh = plsc.VectorSubcoreMesh(core_axis_name="c", subcore_axis_name="s")

    @functools.partial(
        pl.kernel,
        mesh=mesh,
        out_type=jax.ShapeDtypeStruct((O, DT, BT, 8, 128), jnp.float32),
        scratch_types=[
            pltpu.VMEM((O, 128), jnp.int32),
            pltpu.VMEM((DTH * 8, V), jnp.float32),
            pltpu.VMEM((DTH, 8, 128), jnp.float32),
            pltpu.VMEM((DTH, 8, 128), jnp.float32),
            pltpu.SemaphoreType.DMA,
            pltpu.SemaphoreType.DMA,
        ],
        compiler_params=pltpu.CompilerParams(
            use_tc_tiling_on_sc=False, needs_layout_passes=False),
    )
    def k(tableT_hbm, idxT_hbm, out_hbm,
          idx_v, tbl_v, tile0, tile1, sem_w0, sem_w1):
        tiles = (tile0, tile1)
        sems = (sem_w0, sem_w1)

        wid = lax.axis_index("s") * num_cores + lax.axis_index("c")

        tbl_rows = [tbl_v.at[i] for i in range(DTH * 8)]

        def fill_tiles(b, d1):
            ivs = [idx_v[d1, pl.ds(lb * _LANE, _LANE)] for lb in range(nlb)]
            prev = None
            for dt in range(DTH):
                for lb in range(nlb):
                    cur = []
                    for r in range(8):
                        cur.append(
                            plsc.load_gather(tbl_rows[dt * 8 + r], [ivs[lb]]))
                        if prev is not None:
                            pdt, plb, pvals = prev
                            tiles[b][pdt, r, pl.ds(plb * _LANE, _LANE)] = \
                                pvals[r]
                    prev = (dt, lb, cur)
            pdt, plb, pvals = prev
            for r in range(8):
                tiles[b][pdt, r, pl.ds(plb * _LANE, _LANE)] = pvals[r]

        def write_copy(b, d1, hg, bt):
            return pltpu.make_async_copy(
                tiles[b], out_hbm.at[d1, pl.ds(hg * DTH, DTH), bt], sems[b])

        def body_hg(hg, carry):
            pltpu.sync_copy(tableT_hbm.at[pl.ds(hg * DTH * 8, DTH * 8)],
                            tbl_v)

            def body_bt(j, carry2):
                bt = wid * bt_per_w + j
                pltpu.sync_copy(idxT_hbm.at[:, pl.ds(bt * 128, 128)], idx_v)

                for b in range(2):
                    fill_tiles(b, b)
                    write_copy(b, b, hg, bt).start()

                def body_d1(t, carry3):
                    for b in range(2):
                        d1 = 2 * t + b
                        write_copy(b, d1 - 2, hg, bt).wait()
                        fill_tiles(b, d1)
                        write_copy(b, d1, hg, bt).start()
                    return carry3

                lax.fori_loop(1, O // 2, body_d1, 0)

                for b in range(2):
                    write_copy(b, O - 2 + b, hg, bt).wait()
                return carry2

            lax.fori_loop(0, bt_per_w, body_bt, 0)
            return carry

        lax.fori_loop(0, HG, body_hg, 0)

    return k


def kernel(overlap, scene, embed_table):
    B, O = overlap.shape
    V, D = embed_table.shape
    idx_T = overlap.T.astype(jnp.int32)
    table_T = embed_table.T

    info = plsc.get_sparse_core_info()
    n_workers = info.num_cores * info.num_subcores

    k = _gather_kernel(B, O, D, V, n_workers, info.num_cores)
    out5 = k(table_T, idx_T)
    return out5.transpose((2, 4, 0, 1, 3)).reshape(B, O, D)
